# baseline (speedup 1.0000x reference)
import functools

import jax
import jax.numpy as jnp
from jax import lax
from jax.experimental import pallas as pl
from jax.experimental.pallas import tpu as pltpu
from jax.experimental.pallas import tpu_sc as plsc

_N = 50000
_D = 256
_H = 128
_G = 64

_BLK = 5000
_NBLK = _N // _BLK

_KA = 3
_S = _KA * _BLK
_NB2 = _NBLK - _KA

_NEG_INF = float("-inf")


def _mlp_logits(x_ref, w1_ref, b1_ref, w2_ref, b2_ref):
    h = lax.dot_general(w1_ref[...], x_ref[...],
                        (((1,), (1,)), ((), ())),
                        preferred_element_type=jnp.float32)
    h = jnp.maximum(h + b1_ref[...], 0.0)
    lg = lax.dot_general(w2_ref[...], h,
                         (((1,), (0,)), ((), ())),
                         preferred_element_type=jnp.float32)
    return lg + b2_ref[...]


def _body_a(x_ref, w1_ref, b1_ref, w2_ref, b2_ref, lg_ref, stats_ref,
            m_s, z_s):
    i = pl.program_id(0)

    @pl.when(i == 0)
    def _init():
        m_s[...] = jnp.full((1, 128), _NEG_INF, jnp.float32)
        z_s[...] = jnp.zeros((1, 128), jnp.float32)

    lg = _mlp_logits(x_ref, w1_ref, b1_ref, w2_ref, b2_ref)
    lg_ref[...] = lg.reshape(1, 1, _BLK)

    m_old = m_s[0, 0]
    m_new = jnp.maximum(m_old, jnp.max(lg))
    w_u = jnp.exp(lg - m_new)
    resc = jnp.exp(m_s[...] - m_new)
    z_s[...] = z_s[...] * resc + jnp.sum(w_u)
    m_s[...] = jnp.full((1, 128), m_new, jnp.float32)

    @pl.when(i == _KA - 1)
    def _finish():
        stats_ref[...] = jnp.concatenate(
            [jnp.full((1, 16), m_new, jnp.float32),
             jnp.full((1, 16), z_s[0, 0], jnp.float32)], axis=0)


def _stage1a(x, w1, b1c, w2, b2c):
    return pl.pallas_call(
        _body_a,
        grid=(_KA,),
        in_specs=[
            pl.BlockSpec((_BLK, _D), lambda i: (i, 0)),
            pl.BlockSpec((_H, _D), lambda i: (0, 0)),
            pl.BlockSpec((_H, 1), lambda i: (0, 0)),
            pl.BlockSpec((1, _H), lambda i: (0, 0)),
            pl.BlockSpec((1, 1), lambda i: (0, 0)),
        ],
        out_specs=[
            pl.BlockSpec((1, 1, _BLK), lambda i: (i, 0, 0)),
            pl.BlockSpec((2, 16), lambda i: (0, 0)),
        ],
        out_shape=[
            jax.ShapeDtypeStruct((_KA, 1, _BLK), jnp.float32),
            jax.ShapeDtypeStruct((2, 16), jnp.float32),
        ],
        scratch_shapes=[
            pltpu.VMEM((1, 128), jnp.float32),
            pltpu.VMEM((1, 128), jnp.float32),
        ],
    )(x, w1, b1c, w2, b2c)


def _body_b(x_ref, bi_ref, w1_ref, b1_ref, w2_ref, b2_ref, stats_a_ref,
            stats_ref, tcout_ref, m_s, z_s, acc):
    i = pl.program_id(0)

    @pl.when(i == 0)
    def _init():
        m_s[...] = jnp.full((1, 128), stats_a_ref[0, 0], jnp.float32)
        z_s[...] = jnp.full((1, 128), stats_a_ref[1, 0], jnp.float32)
        acc[...] = jnp.zeros((_G, _D), jnp.float32)

    lg = _mlp_logits(x_ref, w1_ref, b1_ref, w2_ref, b2_ref)

    m_old = m_s[0, 0]
    m_new = jnp.maximum(m_old, jnp.max(lg))
    w_u = jnp.exp(lg - m_new)
    resc = jnp.exp(m_s[...] - m_new)
    z_s[...] = z_s[...] * resc + jnp.sum(w_u)
    m_s[...] = jnp.full((1, 128), m_new, jnp.float32)

    g = lax.broadcasted_iota(jnp.int32, (_G, 1), 0)
    sel_w = jnp.where(bi_ref[...].reshape(1, _BLK) == g, w_u, 0.0)
    part = lax.dot_general(sel_w, x_ref[...],
                           (((1,), (0,)), ((), ())),
                           preferred_element_type=jnp.float32)
    acc[...] = acc[...] * resc[0, 0] + part

    @pl.when(i == _NB2 - 1)
    def _finish():
        inv_z = 1.0 / z_s[0, 0]
        stats_ref[...] = jnp.concatenate(
            [jnp.full((1, 16), m_new, jnp.float32),
             jnp.full((1, 16), inv_z, jnp.float32)], axis=0)
        tcout_ref[...] = acc[...] * inv_z


def _stage1b(x, bi2, w1, b1c, w2, b2c, stats_a):
    return pl.pallas_call(
        _body_b,
        grid=(_NB2,),
        in_specs=[
            pl.BlockSpec((_BLK, _D), lambda i: (i + _KA, 0)),
            pl.BlockSpec((1, 1, _BLK), lambda i: (i + _KA, 0, 0)),
            pl.BlockSpec((_H, _D), lambda i: (0, 0)),
            pl.BlockSpec((_H, 1), lambda i: (0, 0)),
            pl.BlockSpec((1, _H), lambda i: (0, 0)),
            pl.BlockSpec((1, 1), lambda i: (0, 0)),
            pl.BlockSpec((2, 16), lambda i: (0, 0)),
        ],
        out_specs=[
            pl.BlockSpec((2, 16), lambda i: (0, 0)),
            pl.BlockSpec((_G, _D), lambda i: (0, 0)),
        ],
        out_shape=[
            jax.ShapeDtypeStruct((2, 16), jnp.float32),
            jax.ShapeDtypeStruct((_G, _D), jnp.float32),
        ],
        scratch_shapes=[
            pltpu.VMEM((1, 128), jnp.float32),
            pltpu.VMEM((1, 128), jnp.float32),
            pltpu.VMEM((_G, _D), jnp.float32),
        ],
    )(x, bi2, w1, b1c, w2, b2c, stats_a)


_TROWS = 120
_NTILES = _S // _TROWS
_NC = 2
_NS = 16
_NW = _NC * _NS


def _sc_pool_body(x_hbm, lg_hbm, bi_hbm, stats_hbm, out_hbm,
                  x0, l0, b0, x1, l1, b1, stats_v, acc, sem0, sem1):
    c = lax.axis_index("c")
    s = lax.axis_index("s")
    wid = s * _NC + c

    def _zero(i, _):
        for cc in range(_D // 16):
            acc[pl.ds(i * _D + cc * 16, 16)] = jnp.zeros((16,), jnp.float32)
        return 0

    lax.fori_loop(0, _G, _zero, 0)

    pltpu.sync_copy(stats_hbm, stats_v)
    m_v = stats_v[0, :]

    nt = (_NTILES - wid + _NW - 1) // _NW
    banks = ((x0, l0, b0, sem0), (x1, l1, b1, sem1))

    def _start(k, bank):
        xb, lb, bb, sm = bank
        r0 = (wid + k * _NW) * _TROWS
        pltpu.async_copy(x_hbm.at[pl.ds(r0, _TROWS), :], xb, sm)
        pltpu.async_copy(lg_hbm.at[pl.ds(r0, _TROWS)], lb, sm)
        pltpu.async_copy(bi_hbm.at[pl.ds(r0, _TROWS)], bb, sm)

    def _wait(k, bank):
        xb, lb, bb, sm = bank
        r0 = (wid + k * _NW) * _TROWS
        pltpu.make_async_copy(x_hbm.at[pl.ds(r0, _TROWS), :], xb, sm).wait()
        pltpu.make_async_copy(lg_hbm.at[pl.ds(r0, _TROWS)], lb, sm).wait()
        pltpu.make_async_copy(bi_hbm.at[pl.ds(r0, _TROWS)], bb, sm).wait()

    def _process(bank):
        xb, lb, bb, _ = bank

        def _group(j, _):
            lv = lb[pl.ds(j * 16, 16)]
            wvec = jnp.exp(lv - m_v)
            bvec = bb[pl.ds(j * 16, 16)]
            uniform = bvec[0] == bvec[15]

            def _fast():
                accs = [jnp.zeros((16,), jnp.float32)
                        for _ in range(_D // 16)]
                for lane in range(16):
                    wb = jnp.full((16,), wvec[lane], jnp.float32)
                    r = j * 16 + lane
                    for cc in range(_D // 16):
                        accs[cc] = accs[cc] + wb * xb[r, pl.ds(cc * 16, 16)]
                base = bvec[0] * _D
                for cc in range(_D // 16):
                    plsc.addupdate(acc.at[pl.ds(base + cc * 16, 16)],
                                   accs[cc])

            def _slow():
                for lane in range(16):
                    wb = jnp.full((16,), wvec[lane], jnp.float32)
                    r = j * 16 + lane
                    base = bvec[lane] * _D
                    for cc in range(_D // 16):
                        xv = xb[r, pl.ds(cc * 16, 16)]
                        plsc.addupdate(acc.at[pl.ds(base + cc * 16, 16)],
                                       wb * xv)

            lax.cond(uniform, _fast, _slow)
            return 0

        lax.fori_loop(0, _TROWS // 16, _group, 0)

    _start(0, banks[0])

    def _pair(p, _):
        k0 = 2 * p
        k1 = k0 + 1
        _wait(k0, banks[0])

        @pl.when(k1 < nt)
        def _():
            _start(k1, banks[1])

        _process(banks[0])

        @pl.when(k1 < nt)
        def _():
            _wait(k1, banks[1])

            @pl.when(k1 + 1 < nt)
            def _():
                _start(k1 + 1, banks[0])

            _process(banks[1])

        return 0

    lax.fori_loop(0, (nt + 1) // 2, _pair, 0)

    pltpu.sync_copy(acc, out_hbm.at[wid])


_STAGE2_CACHE = []


def _stage2(x, lg_flat, batch_index, stats):
    if not _STAGE2_CACHE:
        _STAGE2_CACHE.append(functools.partial(
            pl.kernel,
            mesh=plsc.VectorSubcoreMesh(core_axis_name="c",
                                        subcore_axis_name="s"),
            out_type=jax.ShapeDtypeStruct((_NW, _G * _D), jnp.float32),
            scratch_types=[
                pltpu.VMEM((_TROWS, _D), jnp.float32),
                pltpu.VMEM((_TROWS,), jnp.float32),
                pltpu.VMEM((_TROWS,), jnp.int32),
                pltpu.VMEM((_TROWS, _D), jnp.float32),
                pltpu.VMEM((_TROWS,), jnp.float32),
                pltpu.VMEM((_TROWS,), jnp.int32),
                pltpu.VMEM((2, 16), jnp.float32),
                pltpu.VMEM((_G * _D,), jnp.float32),
                pltpu.SemaphoreType.DMA,
                pltpu.SemaphoreType.DMA,
            ],
        )(_sc_pool_body))
    return _STAGE2_CACHE[0](x, lg_flat, batch_index, stats)


def kernel(node_features, batch_index, W1, b1, W2, b2):
    b1c = b1.reshape(_H, 1)
    b2c = b2.reshape(1, 1)
    bi2 = batch_index.reshape(_NBLK, 1, _BLK)
    logits_a, stats_a = _stage1a(node_features, W1, b1c, W2, b2c)
    stats_b, tc_out = _stage1b(node_features, bi2, W1, b1c, W2, b2c, stats_a)
    lg_flat = logits_a.reshape(-1)
    partials = _stage2(node_features, lg_flat, batch_index, stats_a)
    scale = jnp.exp(stats_a[0, 0] - stats_b[0, 0]) * stats_b[1, 0]
    return partials.sum(axis=0).reshape(_G, _D) * scale + tc_out


# speedup vs baseline: 1.0204x; 1.0204x over previous
import functools

import jax
import jax.numpy as jnp
from jax import lax
from jax.experimental import pallas as pl
from jax.experimental.pallas import tpu as pltpu
from jax.experimental.pallas import tpu_sc as plsc

_N = 50000
_D = 256
_H = 128
_G = 64

_BLK = 5000
_NBLK = _N // _BLK

_KA = 2
_S = _KA * _BLK
_NB2 = _NBLK - _KA

_NEG_INF = float("-inf")


def _mlp_logits(x_ref, w1_ref, b1_ref, w2_ref, b2_ref):
    h = lax.dot_general(w1_ref[...], x_ref[...],
                        (((1,), (1,)), ((), ())),
                        preferred_element_type=jnp.float32)
    h = jnp.maximum(h + b1_ref[...], 0.0)
    lg = lax.dot_general(w2_ref[...], h,
                         (((1,), (0,)), ((), ())),
                         preferred_element_type=jnp.float32)
    return lg + b2_ref[...]


def _body_a(x_ref, w1_ref, b1_ref, w2_ref, b2_ref, lg_ref, stats_ref,
            m_s, z_s):
    i = pl.program_id(0)

    @pl.when(i == 0)
    def _init():
        m_s[...] = jnp.full((1, 128), _NEG_INF, jnp.float32)
        z_s[...] = jnp.zeros((1, 128), jnp.float32)

    lg = _mlp_logits(x_ref, w1_ref, b1_ref, w2_ref, b2_ref)
    lg_ref[...] = lg.reshape(1, 1, _BLK)

    m_old = m_s[0, 0]
    m_new = jnp.maximum(m_old, jnp.max(lg))
    w_u = jnp.exp(lg - m_new)
    resc = jnp.exp(m_s[...] - m_new)
    z_s[...] = z_s[...] * resc + jnp.sum(w_u)
    m_s[...] = jnp.full((1, 128), m_new, jnp.float32)

    @pl.when(i == _KA - 1)
    def _finish():
        stats_ref[...] = jnp.concatenate(
            [jnp.full((1, 16), m_new, jnp.float32),
             jnp.full((1, 16), z_s[0, 0], jnp.float32)], axis=0)


def _stage1a(x, w1, b1c, w2, b2c):
    return pl.pallas_call(
        _body_a,
        grid=(_KA,),
        in_specs=[
            pl.BlockSpec((_BLK, _D), lambda i: (i, 0)),
            pl.BlockSpec((_H, _D), lambda i: (0, 0)),
            pl.BlockSpec((_H, 1), lambda i: (0, 0)),
            pl.BlockSpec((1, _H), lambda i: (0, 0)),
            pl.BlockSpec((1, 1), lambda i: (0, 0)),
        ],
        out_specs=[
            pl.BlockSpec((1, 1, _BLK), lambda i: (i, 0, 0)),
            pl.BlockSpec((2, 16), lambda i: (0, 0)),
        ],
        out_shape=[
            jax.ShapeDtypeStruct((_KA, 1, _BLK), jnp.float32),
            jax.ShapeDtypeStruct((2, 16), jnp.float32),
        ],
        scratch_shapes=[
            pltpu.VMEM((1, 128), jnp.float32),
            pltpu.VMEM((1, 128), jnp.float32),
        ],
    )(x, w1, b1c, w2, b2c)


def _body_b(x_ref, bi_ref, w1_ref, b1_ref, w2_ref, b2_ref, stats_a_ref,
            stats_ref, tcout_ref, m_s, z_s, acc):
    i = pl.program_id(0)

    @pl.when(i == 0)
    def _init():
        m_s[...] = jnp.full((1, 128), stats_a_ref[0, 0], jnp.float32)
        z_s[...] = jnp.full((1, 128), stats_a_ref[1, 0], jnp.float32)
        acc[...] = jnp.zeros((_G, _D), jnp.float32)

    lg = _mlp_logits(x_ref, w1_ref, b1_ref, w2_ref, b2_ref)

    m_old = m_s[0, 0]
    m_new = jnp.maximum(m_old, jnp.max(lg))
    w_u = jnp.exp(lg - m_new)
    resc = jnp.exp(m_s[...] - m_new)
    z_s[...] = z_s[...] * resc + jnp.sum(w_u)
    m_s[...] = jnp.full((1, 128), m_new, jnp.float32)

    g = lax.broadcasted_iota(jnp.int32, (_G, 1), 0)
    sel_w = jnp.where(bi_ref[...].reshape(1, _BLK) == g, w_u, 0.0)
    part = lax.dot_general(sel_w, x_ref[...],
                           (((1,), (0,)), ((), ())),
                           preferred_element_type=jnp.float32)
    acc[...] = acc[...] * resc[0, 0] + part

    @pl.when(i == _NB2 - 1)
    def _finish():
        inv_z = 1.0 / z_s[0, 0]
        stats_ref[...] = jnp.concatenate(
            [jnp.full((1, 16), m_new, jnp.float32),
             jnp.full((1, 16), inv_z, jnp.float32)], axis=0)
        tcout_ref[...] = acc[...] * inv_z


def _stage1b(x, bi2, w1, b1c, w2, b2c, stats_a):
    return pl.pallas_call(
        _body_b,
        grid=(_NB2,),
        in_specs=[
            pl.BlockSpec((_BLK, _D), lambda i: (i + _KA, 0)),
            pl.BlockSpec((1, 1, _BLK), lambda i: (i + _KA, 0, 0)),
            pl.BlockSpec((_H, _D), lambda i: (0, 0)),
            pl.BlockSpec((_H, 1), lambda i: (0, 0)),
            pl.BlockSpec((1, _H), lambda i: (0, 0)),
            pl.BlockSpec((1, 1), lambda i: (0, 0)),
            pl.BlockSpec((2, 16), lambda i: (0, 0)),
        ],
        out_specs=[
            pl.BlockSpec((2, 16), lambda i: (0, 0)),
            pl.BlockSpec((_G, _D), lambda i: (0, 0)),
        ],
        out_shape=[
            jax.ShapeDtypeStruct((2, 16), jnp.float32),
            jax.ShapeDtypeStruct((_G, _D), jnp.float32),
        ],
        scratch_shapes=[
            pltpu.VMEM((1, 128), jnp.float32),
            pltpu.VMEM((1, 128), jnp.float32),
            pltpu.VMEM((_G, _D), jnp.float32),
        ],
    )(x, bi2, w1, b1c, w2, b2c, stats_a)


_TROWS = 40
_NTILES = _S // _TROWS
_NC = 2
_NS = 16
_NW = _NC * _NS


def _sc_pool_body(x_hbm, lg_hbm, bi_hbm, stats_hbm, out_hbm,
                  x0, l0, b0, x1, l1, b1, stats_v, acc, sem0, sem1):
    c = lax.axis_index("c")
    s = lax.axis_index("s")
    wid = s * _NC + c

    def _zero(i, _):
        for cc in range(_D // 16):
            acc[pl.ds(i * _D + cc * 16, 16)] = jnp.zeros((16,), jnp.float32)
        return 0

    lax.fori_loop(0, _G, _zero, 0)

    pltpu.sync_copy(stats_hbm, stats_v)
    m_v = stats_v[0, :]

    nt = (_NTILES - wid + _NW - 1) // _NW
    banks = ((x0, l0, b0, sem0), (x1, l1, b1, sem1))

    def _start(k, bank):
        xb, lb, bb, sm = bank
        r0 = (wid + k * _NW) * _TROWS
        pltpu.async_copy(x_hbm.at[pl.ds(r0, _TROWS), :], xb, sm)
        pltpu.async_copy(lg_hbm.at[pl.ds(r0, _TROWS)], lb, sm)
        pltpu.async_copy(bi_hbm.at[pl.ds(r0, _TROWS)], bb, sm)

    def _wait(k, bank):
        xb, lb, bb, sm = bank
        r0 = (wid + k * _NW) * _TROWS
        pltpu.make_async_copy(x_hbm.at[pl.ds(r0, _TROWS), :], xb, sm).wait()
        pltpu.make_async_copy(lg_hbm.at[pl.ds(r0, _TROWS)], lb, sm).wait()
        pltpu.make_async_copy(bi_hbm.at[pl.ds(r0, _TROWS)], bb, sm).wait()

    def _process(bank):
        xb, lb, bb, _ = bank

        def _group(j, _):
            lv = lb[pl.ds(j * 16, 16)]
            wvec = jnp.exp(lv - m_v)
            bvec = bb[pl.ds(j * 16, 16)]
            uniform = bvec[0] == bvec[15]

            def _fast():
                accs = [jnp.zeros((16,), jnp.float32)
                        for _ in range(_D // 16)]
                for lane in range(16):
                    wb = jnp.full((16,), wvec[lane], jnp.float32)
                    r = j * 16 + lane
                    for cc in range(_D // 16):
                        accs[cc] = accs[cc] + wb * xb[r, pl.ds(cc * 16, 16)]
                base = bvec[0] * _D
                for cc in range(_D // 16):
                    plsc.addupdate(acc.at[pl.ds(base + cc * 16, 16)],
                                   accs[cc])

            def _slow():
                for lane in range(16):
                    wb = jnp.full((16,), wvec[lane], jnp.float32)
                    r = j * 16 + lane
                    base = bvec[lane] * _D
                    for cc in range(_D // 16):
                        xv = xb[r, pl.ds(cc * 16, 16)]
                        plsc.addupdate(acc.at[pl.ds(base + cc * 16, 16)],
                                       wb * xv)

            lax.cond(uniform, _fast, _slow)
            return 0

        lax.fori_loop(0, _TROWS // 16, _group, 0)

    _start(0, banks[0])

    def _pair(p, _):
        k0 = 2 * p
        k1 = k0 + 1
        _wait(k0, banks[0])

        @pl.when(k1 < nt)
        def _():
            _start(k1, banks[1])

        _process(banks[0])

        @pl.when(k1 < nt)
        def _():
            _wait(k1, banks[1])

            @pl.when(k1 + 1 < nt)
            def _():
                _start(k1 + 1, banks[0])

            _process(banks[1])

        return 0

    lax.fori_loop(0, (nt + 1) // 2, _pair, 0)

    pltpu.sync_copy(acc, out_hbm.at[wid])


_STAGE2_CACHE = []


def _stage2(x, lg_flat, batch_index, stats):
    if not _STAGE2_CACHE:
        _STAGE2_CACHE.append(functools.partial(
            pl.kernel,
            mesh=plsc.VectorSubcoreMesh(core_axis_name="c",
                                        subcore_axis_name="s"),
            out_type=jax.ShapeDtypeStruct((_NW, _G * _D), jnp.float32),
            scratch_types=[
                pltpu.VMEM((_TROWS, _D), jnp.float32),
                pltpu.VMEM((_TROWS,), jnp.float32),
                pltpu.VMEM((_TROWS,), jnp.int32),
                pltpu.VMEM((_TROWS, _D), jnp.float32),
                pltpu.VMEM((_TROWS,), jnp.float32),
                pltpu.VMEM((_TROWS,), jnp.int32),
                pltpu.VMEM((2, 16), jnp.float32),
                pltpu.VMEM((_G * _D,), jnp.float32),
                pltpu.SemaphoreType.DMA,
                pltpu.SemaphoreType.DMA,
            ],
        )(_sc_pool_body))
    return _STAGE2_CACHE[0](x, lg_flat, batch_index, stats)


def kernel(node_features, batch_index, W1, b1, W2, b2):
    b1c = b1.reshape(_H, 1)
    b2c = b2.reshape(1, 1)
    bi2 = batch_index.reshape(_NBLK, 1, _BLK)
    logits_a, stats_a = _stage1a(node_features, W1, b1c, W2, b2c)
    stats_b, tc_out = _stage1b(node_features, bi2, W1, b1c, W2, b2c, stats_a)
    lg_flat = logits_a.reshape(-1)
    partials = _stage2(node_features, lg_flat, batch_index, stats_a)
    scale = jnp.exp(stats_a[0, 0] - stats_b[0, 0]) * stats_b[1, 0]
    return partials.sum(axis=0).reshape(_G, _D) * scale + tc_out


# speedup vs baseline: 1.0413x; 1.0205x over previous
import functools

import jax
import jax.numpy as jnp
from jax import lax
from jax.experimental import pallas as pl
from jax.experimental.pallas import tpu as pltpu
from jax.experimental.pallas import tpu_sc as plsc

_N = 50000
_D = 256
_H = 128
_G = 64

_BLK = 5000
_NBLK = _N // _BLK

_KA = 2
_S = _KA * _BLK
_BLKB = 10000
_NB2 = (_N - _S) // _BLKB
_OFFB = _S // _BLKB

_NEG_INF = float("-inf")


def _mlp_logits(x_ref, w1_ref, b1_ref, w2_ref, b2_ref):
    h = lax.dot_general(w1_ref[...], x_ref[...],
                        (((1,), (1,)), ((), ())),
                        preferred_element_type=jnp.float32)
    h = jnp.maximum(h + b1_ref[...], 0.0)
    lg = lax.dot_general(w2_ref[...], h,
                         (((1,), (0,)), ((), ())),
                         preferred_element_type=jnp.float32)
    return lg + b2_ref[...]


def _body_a(x_ref, w1_ref, b1_ref, w2_ref, b2_ref, lg_ref, stats_ref,
            m_s, z_s):
    i = pl.program_id(0)

    @pl.when(i == 0)
    def _init():
        m_s[...] = jnp.full((1, 128), _NEG_INF, jnp.float32)
        z_s[...] = jnp.zeros((1, 128), jnp.float32)

    lg = _mlp_logits(x_ref, w1_ref, b1_ref, w2_ref, b2_ref)
    lg_ref[...] = lg.reshape(1, 1, _BLK)

    m_old = m_s[0, 0]
    m_new = jnp.maximum(m_old, jnp.max(lg))
    w_u = jnp.exp(lg - m_new)
    resc = jnp.exp(m_s[...] - m_new)
    z_s[...] = z_s[...] * resc + jnp.sum(w_u)
    m_s[...] = jnp.full((1, 128), m_new, jnp.float32)

    @pl.when(i == _KA - 1)
    def _finish():
        stats_ref[...] = jnp.concatenate(
            [jnp.full((1, 16), m_new, jnp.float32),
             jnp.full((1, 16), z_s[0, 0], jnp.float32)], axis=0)


def _stage1a(x, w1, b1c, w2, b2c):
    return pl.pallas_call(
        _body_a,
        grid=(_KA,),
        in_specs=[
            pl.BlockSpec((_BLK, _D), lambda i: (i, 0)),
            pl.BlockSpec((_H, _D), lambda i: (0, 0)),
            pl.BlockSpec((_H, 1), lambda i: (0, 0)),
            pl.BlockSpec((1, _H), lambda i: (0, 0)),
            pl.BlockSpec((1, 1), lambda i: (0, 0)),
        ],
        out_specs=[
            pl.BlockSpec((1, 1, _BLK), lambda i: (i, 0, 0)),
            pl.BlockSpec((2, 16), lambda i: (0, 0)),
        ],
        out_shape=[
            jax.ShapeDtypeStruct((_KA, 1, _BLK), jnp.float32),
            jax.ShapeDtypeStruct((2, 16), jnp.float32),
        ],
        scratch_shapes=[
            pltpu.VMEM((1, 128), jnp.float32),
            pltpu.VMEM((1, 128), jnp.float32),
        ],
    )(x, w1, b1c, w2, b2c)


def _body_b(x_ref, bi_ref, w1_ref, b1_ref, w2_ref, b2_ref, stats_a_ref,
            stats_ref, tcout_ref, m_s, z_s, acc):
    i = pl.program_id(0)

    @pl.when(i == 0)
    def _init():
        m_s[...] = jnp.full((1, 128), stats_a_ref[0, 0], jnp.float32)
        z_s[...] = jnp.full((1, 128), stats_a_ref[1, 0], jnp.float32)
        acc[...] = jnp.zeros((_G, _D), jnp.float32)

    lg = _mlp_logits(x_ref, w1_ref, b1_ref, w2_ref, b2_ref)

    m_old = m_s[0, 0]
    m_new = jnp.maximum(m_old, jnp.max(lg))
    w_u = jnp.exp(lg - m_new)
    resc = jnp.exp(m_s[...] - m_new)
    z_s[...] = z_s[...] * resc + jnp.sum(w_u)
    m_s[...] = jnp.full((1, 128), m_new, jnp.float32)

    g = lax.broadcasted_iota(jnp.int32, (_G, 1), 0)
    sel_w = jnp.where(bi_ref[...].reshape(1, _BLKB) == g, w_u, 0.0)
    part = lax.dot_general(sel_w, x_ref[...],
                           (((1,), (0,)), ((), ())),
                           preferred_element_type=jnp.float32)
    acc[...] = acc[...] * resc[0, 0] + part

    @pl.when(i == _NB2 - 1)
    def _finish():
        inv_z = 1.0 / z_s[0, 0]
        stats_ref[...] = jnp.concatenate(
            [jnp.full((1, 16), m_new, jnp.float32),
             jnp.full((1, 16), inv_z, jnp.float32)], axis=0)
        tcout_ref[...] = acc[...] * inv_z


def _stage1b(x, bi2, w1, b1c, w2, b2c, stats_a):
    return pl.pallas_call(
        _body_b,
        grid=(_NB2,),
        in_specs=[
            pl.BlockSpec((_BLKB, _D), lambda i: (i + _OFFB, 0)),
            pl.BlockSpec((1, 1, _BLKB), lambda i: (i + _OFFB, 0, 0)),
            pl.BlockSpec((_H, _D), lambda i: (0, 0)),
            pl.BlockSpec((_H, 1), lambda i: (0, 0)),
            pl.BlockSpec((1, _H), lambda i: (0, 0)),
            pl.BlockSpec((1, 1), lambda i: (0, 0)),
            pl.BlockSpec((2, 16), lambda i: (0, 0)),
        ],
        out_specs=[
            pl.BlockSpec((2, 16), lambda i: (0, 0)),
            pl.BlockSpec((_G, _D), lambda i: (0, 0)),
        ],
        out_shape=[
            jax.ShapeDtypeStruct((2, 16), jnp.float32),
            jax.ShapeDtypeStruct((_G, _D), jnp.float32),
        ],
        scratch_shapes=[
            pltpu.VMEM((1, 128), jnp.float32),
            pltpu.VMEM((1, 128), jnp.float32),
            pltpu.VMEM((_G, _D), jnp.float32),
        ],
    )(x, bi2, w1, b1c, w2, b2c, stats_a)


_TROWS = 40
_NTILES = _S // _TROWS
_NC = 2
_NS = 16
_NW = _NC * _NS


def _sc_pool_body(x_hbm, lg_hbm, bi_hbm, stats_hbm, out_hbm,
                  x0, l0, b0, x1, l1, b1, stats_v, acc, sem0, sem1):
    c = lax.axis_index("c")
    s = lax.axis_index("s")
    wid = s * _NC + c

    def _zero(i, _):
        for cc in range(_D // 16):
            acc[pl.ds(i * _D + cc * 16, 16)] = jnp.zeros((16,), jnp.float32)
        return 0

    lax.fori_loop(0, _G, _zero, 0)

    pltpu.sync_copy(stats_hbm, stats_v)
    m_v = stats_v[0, :]

    nt = (_NTILES - wid + _NW - 1) // _NW
    banks = ((x0, l0, b0, sem0), (x1, l1, b1, sem1))

    def _start(k, bank):
        xb, lb, bb, sm = bank
        r0 = (wid + k * _NW) * _TROWS
        pltpu.async_copy(x_hbm.at[pl.ds(r0, _TROWS), :], xb, sm)
        pltpu.async_copy(lg_hbm.at[pl.ds(r0, _TROWS)], lb, sm)
        pltpu.async_copy(bi_hbm.at[pl.ds(r0, _TROWS)], bb, sm)

    def _wait(k, bank):
        xb, lb, bb, sm = bank
        r0 = (wid + k * _NW) * _TROWS
        pltpu.make_async_copy(x_hbm.at[pl.ds(r0, _TROWS), :], xb, sm).wait()
        pltpu.make_async_copy(lg_hbm.at[pl.ds(r0, _TROWS)], lb, sm).wait()
        pltpu.make_async_copy(bi_hbm.at[pl.ds(r0, _TROWS)], bb, sm).wait()

    def _process(bank):
        xb, lb, bb, _ = bank

        def _group(j, _):
            lv = lb[pl.ds(j * 16, 16)]
            wvec = jnp.exp(lv - m_v)
            bvec = bb[pl.ds(j * 16, 16)]
            uniform = bvec[0] == bvec[15]

            def _fast():
                accs = [jnp.zeros((16,), jnp.float32)
                        for _ in range(_D // 16)]
                for lane in range(16):
                    wb = jnp.full((16,), wvec[lane], jnp.float32)
                    r = j * 16 + lane
                    for cc in range(_D // 16):
                        accs[cc] = accs[cc] + wb * xb[r, pl.ds(cc * 16, 16)]
                base = bvec[0] * _D
                for cc in range(_D // 16):
                    plsc.addupdate(acc.at[pl.ds(base + cc * 16, 16)],
                                   accs[cc])

            def _slow():
                for lane in range(16):
                    wb = jnp.full((16,), wvec[lane], jnp.float32)
                    r = j * 16 + lane
                    base = bvec[lane] * _D
                    for cc in range(_D // 16):
                        xv = xb[r, pl.ds(cc * 16, 16)]
                        plsc.addupdate(acc.at[pl.ds(base + cc * 16, 16)],
                                       wb * xv)

            lax.cond(uniform, _fast, _slow)
            return 0

        lax.fori_loop(0, _TROWS // 16, _group, 0)

    _start(0, banks[0])

    def _pair(p, _):
        k0 = 2 * p
        k1 = k0 + 1
        _wait(k0, banks[0])

        @pl.when(k1 < nt)
        def _():
            _start(k1, banks[1])

        _process(banks[0])

        @pl.when(k1 < nt)
        def _():
            _wait(k1, banks[1])

            @pl.when(k1 + 1 < nt)
            def _():
                _start(k1 + 1, banks[0])

            _process(banks[1])

        return 0

    lax.fori_loop(0, (nt + 1) // 2, _pair, 0)

    pltpu.sync_copy(acc, out_hbm.at[wid])


_STAGE2_CACHE = []


def _stage2(x, lg_flat, batch_index, stats):
    if not _STAGE2_CACHE:
        _STAGE2_CACHE.append(functools.partial(
            pl.kernel,
            mesh=plsc.VectorSubcoreMesh(core_axis_name="c",
                                        subcore_axis_name="s"),
            out_type=jax.ShapeDtypeStruct((_NW, _G * _D), jnp.float32),
            scratch_types=[
                pltpu.VMEM((_TROWS, _D), jnp.float32),
                pltpu.VMEM((_TROWS,), jnp.float32),
                pltpu.VMEM((_TROWS,), jnp.int32),
                pltpu.VMEM((_TROWS, _D), jnp.float32),
                pltpu.VMEM((_TROWS,), jnp.float32),
                pltpu.VMEM((_TROWS,), jnp.int32),
                pltpu.VMEM((2, 16), jnp.float32),
                pltpu.VMEM((_G * _D,), jnp.float32),
                pltpu.SemaphoreType.DMA,
                pltpu.SemaphoreType.DMA,
            ],
        )(_sc_pool_body))
    return _STAGE2_CACHE[0](x, lg_flat, batch_index, stats)


def kernel(node_features, batch_index, W1, b1, W2, b2):
    b1c = b1.reshape(_H, 1)
    b2c = b2.reshape(1, 1)
    bi2 = batch_index.reshape(_N // _BLKB, 1, _BLKB)
    logits_a, stats_a = _stage1a(node_features, W1, b1c, W2, b2c)
    stats_b, tc_out = _stage1b(node_features, bi2, W1, b1c, W2, b2c, stats_a)
    lg_flat = logits_a.reshape(-1)
    partials = _stage2(node_features, lg_flat, batch_index, stats_a)
    scale = jnp.exp(stats_a[0, 0] - stats_b[0, 0]) * stats_b[1, 0]
    return partials.sum(axis=0).reshape(_G, _D) * scale + tc_out
